# baseline (speedup 1.0000x reference)
import jax
import jax.numpy as jnp
from jax.experimental import pallas as pl
from jax.experimental.pallas import tpu as pltpu

_SC = 512


def _dot(a, b, dims):
    return jax.lax.dot_general(a, b, (dims, ((), ())),
                               preferred_element_type=jnp.float32)


def _fused(x_ref, p_ref, ls_ref, amp_ref, wv_ref, wo_ref, out_ref,
           aff_ref, c_ref, g_ref, m_ref, *, nt, ns, kk):
    t = pl.program_id(0)
    n1 = nt

    @pl.when(t < n1)
    def _p1():
        b = t // ns
        x = x_ref[0]
        P = p_ref[...]
        ls = ls_ref[0]
        amp = amp_ref[0]
        x2 = jnp.sum(x * x, axis=1, keepdims=True)
        p2 = jnp.sum(P * P, axis=1)
        xp = _dot(x, P, ((1,), (1,)))
        d2 = jnp.maximum(x2 + p2[None, :] - 2.0 * xp, 0.0)
        inv = 1.0 / (2.0 * jnp.exp(2.0 * ls) + 1e-8)
        aff = amp[None, :] * jnp.exp(-d2 * inv[None, :])
        aff_ref[pl.ds(t * _SC, _SC), :] = aff
        c_part = _dot(aff, x, ((0,), (0,)))
        g_part = jnp.sum(aff, axis=0, keepdims=True)

        @pl.when(t % ns == 0)
        def _init():
            c_ref[pl.ds(b * kk, kk), :] = c_part
            g_ref[pl.ds(b, 1), :] = g_part

        @pl.when(t % ns != 0)
        def _acc():
            c_ref[pl.ds(b * kk, kk), :] += c_part
            g_ref[pl.ds(b, 1), :] += g_part

    @pl.when(t == n1)
    def _fold():
        cv = _dot(c_ref[...], wv_ref[...], ((1,), (0,)))
        m_ref[...] = _dot(cv, wo_ref[...], ((1,), (0,)))

    @pl.when(t > n1)
    def _p2():
        q = t - n1 - 1
        b = q // ns
        aff = aff_ref[pl.ds(q * _SC, _SC), :]
        g_row = g_ref[pl.ds(b, 1), :]
        m = m_ref[pl.ds(b * kk, kk), :]
        denom = jnp.sum(aff * g_row, axis=1, keepdims=True) + 1e-8
        out_ref[0] = _dot(aff / denom, m, ((1,), (0,)))


def kernel(token_embeddings, positions, log_scales, amplitudes, Wv, Wo):
    B, S, D = token_embeddings.shape
    K = positions.shape[0]
    nS = S // _SC
    nt = B * nS
    ls2 = log_scales.reshape(1, K).astype(jnp.float32)
    amp2 = amplitudes.reshape(1, K).astype(jnp.float32)

    import functools

    def x_idx(t):
        q = jnp.minimum(t, nt - 1)
        return (q // nS, q % nS, 0)

    def out_idx(t):
        q = jnp.clip(t - nt - 1, 0, nt - 1)
        return (q // nS, q % nS, 0)

    return pl.pallas_call(
        functools.partial(_fused, nt=nt, ns=nS, kk=K),
        grid=(2 * nt + 1,),
        in_specs=[
            pl.BlockSpec((1, _SC, D), x_idx),
            pl.BlockSpec((K, D), lambda t: (0, 0)),
            pl.BlockSpec((1, K), lambda t: (0, 0)),
            pl.BlockSpec((1, K), lambda t: (0, 0)),
            pl.BlockSpec((D, D), lambda t: (0, 0)),
            pl.BlockSpec((D, D), lambda t: (0, 0)),
        ],
        out_specs=pl.BlockSpec((1, _SC, D), out_idx),
        out_shape=jax.ShapeDtypeStruct((B, S, D), jnp.float32),
        scratch_shapes=[
            pltpu.VMEM((B * S, K), jnp.float32),
            pltpu.VMEM((B * K, D), jnp.float32),
            pltpu.VMEM((B, K), jnp.float32),
            pltpu.VMEM((B * K, D), jnp.float32),
        ],
        compiler_params=pltpu.CompilerParams(
            dimension_semantics=("arbitrary",),
        ),
    )(token_embeddings, positions, ls2, amp2, Wv, Wo)


# speedup vs baseline: 1.1797x; 1.1797x over previous
import jax
import jax.numpy as jnp
from jax.experimental import pallas as pl
from jax.experimental.pallas import tpu as pltpu

_SC = 1024


def _dot(a, b, dims):
    return jax.lax.dot_general(a, b, (dims, ((), ())),
                               preferred_element_type=jnp.float32)


def _fused(x_ref, p_ref, ls_ref, amp_ref, wv_ref, wo_ref, out_ref,
           aff_ref, c_ref, g_ref, m_ref, *, nt, ns, kk):
    t = pl.program_id(0)
    n1 = nt

    @pl.when(t < n1)
    def _p1():
        b = t // ns
        x = x_ref[0]
        P = p_ref[...]
        ls = ls_ref[0]
        amp = amp_ref[0]
        x2 = jnp.sum(x * x, axis=1, keepdims=True)
        p2 = jnp.sum(P * P, axis=1)
        xp = _dot(x, P, ((1,), (1,)))
        d2 = jnp.maximum(x2 + p2[None, :] - 2.0 * xp, 0.0)
        inv = 1.0 / (2.0 * jnp.exp(2.0 * ls) + 1e-8)
        aff = amp[None, :] * jnp.exp(-d2 * inv[None, :])
        aff_ref[pl.ds(t * _SC, _SC), :] = aff
        c_part = _dot(aff, x, ((0,), (0,)))
        g_part = jnp.sum(aff, axis=0, keepdims=True)

        @pl.when(t % ns == 0)
        def _init():
            c_ref[pl.ds(b * kk, kk), :] = c_part
            g_ref[pl.ds(b, 1), :] = g_part

        @pl.when(t % ns != 0)
        def _acc():
            c_ref[pl.ds(b * kk, kk), :] += c_part
            g_ref[pl.ds(b, 1), :] += g_part

    @pl.when(t == n1)
    def _fold():
        cv = _dot(c_ref[...], wv_ref[...], ((1,), (0,)))
        m_ref[...] = _dot(cv, wo_ref[...], ((1,), (0,)))

    @pl.when(t > n1)
    def _p2():
        q = t - n1 - 1
        b = q // ns
        aff = aff_ref[pl.ds(q * _SC, _SC), :]
        g_row = g_ref[pl.ds(b, 1), :]
        m = m_ref[pl.ds(b * kk, kk), :]
        denom = jnp.sum(aff * g_row, axis=1, keepdims=True) + 1e-8
        out_ref[0] = _dot(aff, m, ((1,), (0,))) / denom


def kernel(token_embeddings, positions, log_scales, amplitudes, Wv, Wo):
    B, S, D = token_embeddings.shape
    K = positions.shape[0]
    nS = S // _SC
    nt = B * nS
    ls2 = log_scales.reshape(1, K).astype(jnp.float32)
    amp2 = amplitudes.reshape(1, K).astype(jnp.float32)

    import functools

    def x_idx(t):
        q = jnp.minimum(t, nt - 1)
        return (q // nS, q % nS, 0)

    def out_idx(t):
        q = jnp.clip(t - nt - 1, 0, nt - 1)
        return (q // nS, q % nS, 0)

    return pl.pallas_call(
        functools.partial(_fused, nt=nt, ns=nS, kk=K),
        grid=(2 * nt + 1,),
        in_specs=[
            pl.BlockSpec((1, _SC, D), x_idx),
            pl.BlockSpec((K, D), lambda t: (0, 0)),
            pl.BlockSpec((1, K), lambda t: (0, 0)),
            pl.BlockSpec((1, K), lambda t: (0, 0)),
            pl.BlockSpec((D, D), lambda t: (0, 0)),
            pl.BlockSpec((D, D), lambda t: (0, 0)),
        ],
        out_specs=pl.BlockSpec((1, _SC, D), out_idx),
        out_shape=jax.ShapeDtypeStruct((B, S, D), jnp.float32),
        scratch_shapes=[
            pltpu.VMEM((B * S, K), jnp.float32),
            pltpu.VMEM((B * K, D), jnp.float32),
            pltpu.VMEM((B, K), jnp.float32),
            pltpu.VMEM((B * K, D), jnp.float32),
        ],
        compiler_params=pltpu.CompilerParams(
            dimension_semantics=("arbitrary",),
        ),
    )(token_embeddings, positions, ls2, amp2, Wv, Wo)
